# baseline (speedup 1.0000x reference)
import functools
import jax
import jax.numpy as jnp
from jax.experimental import pallas as pl

F32 = jnp.float32


def _mm(a, b):
    return jax.lax.dot_general(a, b, (((1,), (0,)), ((), ())),
                               preferred_element_type=F32)


def _chunks(m, r):
    out = []
    s = 0
    while s < m:
        out.append((s, min(r, m - s)))
        s += r
    return out


def _bias_relu_mm_body(x_ref, w_ref, b_ref, o_ref):
    for s, r in _chunks(o_ref.shape[0], 1568):
        o_ref[s:s + r, :] = jax.nn.relu(
            _mm(x_ref[s:s + r, :], w_ref[...]) + b_ref[...])


def _shifted_mm_body(offsets, chunk, x_ref, w_ref, b_ref, o_ref):
    for s, r in _chunks(o_ref.shape[1], chunk):
        acc = _mm(x_ref[0, offsets[0] + s:offsets[0] + s + r, :], w_ref[0])
        for t in range(1, len(offsets)):
            acc = acc + _mm(x_ref[0, offsets[t] + s:offsets[t] + s + r, :],
                            w_ref[t])
        o_ref[0, s:s + r, :] = jax.nn.relu(acc + b_ref[...])


def _rb_conv_body(offsets, chunk, x_ref, w_ref, o_ref):
    for s, r in _chunks(o_ref.shape[1], chunk):
        acc = _mm(x_ref[0, offsets[0] + s:offsets[0] + s + r, :], w_ref[0])
        for t in range(1, len(offsets)):
            acc = acc + _mm(x_ref[0, offsets[t] + s:offsets[t] + s + r, :],
                            w_ref[t])
        o_ref[0, s:s + r, :] = acc


def _rb_post_body(h_ref, x_ref, a_ref, w2_ref, o_ref):
    for s, r in _chunks(o_ref.shape[1], 784):
        y = jax.nn.relu((h_ref[0, s:s + r, :] - a_ref[0, 1:2, :])
                        * a_ref[0, 0:1, :] + a_ref[0, 2:3, :])
        o_ref[0, s:s + r, :] = jax.nn.relu(
            x_ref[0, s:s + r, :] + _mm(y, w2_ref[...]))


def _vq_body(x_ref, cbt_ref, cb_ref, c2_ref, o_ref, l_ref):
    lsum = jnp.zeros((128,), F32)
    for s, r in _chunks(o_ref.shape[1], 392):
        x = x_ref[0, s:s + r, :]
        x2 = jnp.sum(x * x, axis=1, keepdims=True)
        dist = x2 + c2_ref[...] - 2.0 * _mm(x, cbt_ref[...])
        q = jnp.argmin(dist, axis=1)
        e = (jax.lax.broadcasted_iota(jnp.int32, dist.shape, 1)
             == q[:, None]).astype(F32)
        zq = jax.lax.dot_general(e, cb_ref[...], (((1,), (0,)), ((), ())),
                                 precision=jax.lax.Precision.HIGHEST,
                                 preferred_element_type=F32)
        o_ref[0, s:s + r, :] = zq
        d = zq - x
        lsum = lsum + jnp.sum(d * d) * jnp.ones((128,), F32)
    l_ref[0, 0, :] = lsum


def _dt1_body(offsets, x_ref, w_ref, b_ref, o_ref):
    for s, r in _chunks(o_ref.shape[1], 424):
        accs = []
        for g in range(4):
            acc = None
            for t in range(4):
                k = g * 4 + t
                p = _mm(x_ref[0, offsets[k] + s:offsets[k] + s + r, :],
                        w_ref[k])
                acc = p if acc is None else acc + p
            accs.append(acc)
        o_ref[0, s:s + r, :] = jax.nn.relu(
            jnp.concatenate(accs, axis=1) + b_ref[...])


def _full(shape):
    nd = len(shape)
    return pl.BlockSpec(shape, lambda i: (0,) * nd)


def _ec1(x, w, b):
    n = x.shape[0]
    xp = jnp.pad(x, ((0, 0), (1, 1), (1, 1), (0, 0)))
    s2d = xp.reshape(n, 113, 2, 113, 2, 3).transpose(0, 1, 3, 2, 4, 5)
    s2d = s2d.reshape(n, 113, 113, 12)
    cat = jnp.concatenate(
        [s2d[:, dy:dy + 112, dx:dx + 112, :] for dy in (0, 1) for dx in (0, 1)],
        axis=-1).reshape(n * 112 * 112, 48)
    wt = w.reshape(128, 3, 2, 2, 2, 2).transpose(2, 4, 3, 5, 1, 0)
    wc = wt.reshape(4, 12, 128).reshape(48, 128)
    out = pl.pallas_call(
        _bias_relu_mm_body,
        grid=(n,),
        in_specs=[pl.BlockSpec((12544, 48), lambda i: (i, 0)),
                  _full((48, 128)), _full((1, 128))],
        out_specs=pl.BlockSpec((12544, 128), lambda i: (i, 0)),
        out_shape=jax.ShapeDtypeStruct((n * 12544, 128), F32),
    )(cat, wc, b.reshape(1, 128))
    return out.reshape(n, 112, 112, 128)


def _ec2(x, w, b):
    n = x.shape[0]
    xp = jnp.pad(x, ((0, 0), (1, 1), (1, 1), (0, 0)))
    s2d = xp.reshape(n, 57, 2, 57, 2, 128).transpose(0, 1, 3, 2, 4, 5)
    xf = s2d.reshape(n, 3249, 512)
    xf = jnp.pad(xf, ((0, 0), (0, 7), (0, 0)))
    wt = w.reshape(256, 128, 2, 2, 2, 2).transpose(2, 4, 3, 5, 1, 0)
    w4 = wt.reshape(4, 512, 256)
    offs = (0, 1, 57, 58)
    out = pl.pallas_call(
        functools.partial(_shifted_mm_body, offs, 456),
        grid=(n,),
        in_specs=[pl.BlockSpec((1, 3256, 512), lambda i: (i, 0, 0)),
                  _full((4, 512, 256)), _full((1, 256))],
        out_specs=pl.BlockSpec((1, 3192, 256), lambda i: (i, 0, 0)),
        out_shape=jax.ShapeDtypeStruct((n, 3192, 256), F32),
    )(xf, w4, b.reshape(1, 256))
    return out.reshape(n, 56, 57, 256)[:, :, :56, :]


def _resblock(x4, w1, g, b, w2):
    n = x4.shape[0]
    xp = jnp.pad(x4, ((0, 0), (1, 1), (1, 1), (0, 0))).reshape(n, 3364, 256)
    xf = jnp.pad(xp, ((0, 0), (0, 124), (0, 0)))
    w9 = w1.transpose(2, 3, 1, 0).reshape(9, 256, 256)
    offs = tuple(dy * 58 + dx for dy in range(3) for dx in range(3))
    h = pl.pallas_call(
        functools.partial(_rb_conv_body, offs, 424),
        grid=(n,),
        in_specs=[pl.BlockSpec((1, 3488, 256), lambda i: (i, 0, 0)),
                  _full((9, 256, 256))],
        out_specs=pl.BlockSpec((1, 3368, 256), lambda i: (i, 0, 0)),
        out_shape=jax.ShapeDtypeStruct((n, 3368, 256), F32),
    )(xf, w9)
    hval = h[:, :3364, :].reshape(n, 58, 58, 256)[:, :56, :56, :]
    hn = jnp.transpose(hval, (0, 3, 1, 2))
    m = jnp.mean(hn, axis=(0, 2, 3))
    v = jnp.var(hn, axis=(0, 2, 3))
    scale = g / jnp.sqrt(v + 1e-5)
    ab = jnp.stack([scale, m, b]).reshape(1, 3, 256)
    hval = hval.reshape(n, 3136, 256)
    xval = x4.reshape(n, 3136, 256)
    out = pl.pallas_call(
        _rb_post_body,
        grid=(n,),
        in_specs=[pl.BlockSpec((1, 3136, 256), lambda i: (i, 0, 0)),
                  pl.BlockSpec((1, 3136, 256), lambda i: (i, 0, 0)),
                  _full((1, 3, 256)), _full((256, 256))],
        out_specs=pl.BlockSpec((1, 3136, 256), lambda i: (i, 0, 0)),
        out_shape=jax.ShapeDtypeStruct((n, 3136, 256), F32),
    )(hval, xval, ab, w2[:, :, 0, 0].T)
    return out.reshape(n, 56, 56, 256)


def _vq(ze, cb):
    n = ze.shape[0]
    x = ze.reshape(n, 3136, 256)
    c2 = jnp.sum(cb * cb, axis=1).reshape(1, 512)
    zq, lp = pl.pallas_call(
        _vq_body,
        grid=(n,),
        in_specs=[pl.BlockSpec((1, 3136, 256), lambda i: (i, 0, 0)),
                  _full((256, 512)), _full((512, 256)), _full((1, 512))],
        out_specs=[pl.BlockSpec((1, 3136, 256), lambda i: (i, 0, 0)),
                   pl.BlockSpec((1, 1, 128), lambda i: (i, 0, 0))],
        out_shape=[jax.ShapeDtypeStruct((n, 3136, 256), F32),
                   jax.ShapeDtypeStruct((n, 1, 128), F32)],
    )(x, cb.T, cb, c2)
    vq_loss = 2.0 * jnp.sum(lp[:, 0, 0]) / (n * 3136 * 256)
    return zq.reshape(n, 56, 56, 256), vq_loss


_T = {0: [(1, 1), (0, 3)], 1: [(1, 2), (2, 0)]}


def _dt1(x4, w, b):
    n = x4.shape[0]
    xp = jnp.pad(x4, ((0, 0), (1, 1), (1, 1), (0, 0))).reshape(n, 3364, 256)
    xf = jnp.pad(xp, ((0, 0), (0, 124), (0, 0)))
    offs, wlist = [], []
    for a in range(2):
        for bb in range(2):
            for (jy, kh) in _T[a]:
                for (jx, kw) in _T[bb]:
                    offs.append(jy * 58 + jx)
                    wlist.append(w[:, :, kh, kw])
    w16 = jnp.stack(wlist)
    b16 = jnp.tile(b, 4).reshape(1, 512)
    out = pl.pallas_call(
        functools.partial(_dt1_body, tuple(offs)),
        grid=(n,),
        in_specs=[pl.BlockSpec((1, 3488, 256), lambda i: (i, 0, 0)),
                  _full((16, 256, 128)), _full((1, 512))],
        out_specs=pl.BlockSpec((1, 3368, 512), lambda i: (i, 0, 0)),
        out_shape=jax.ShapeDtypeStruct((n, 3368, 512), F32),
    )(xf, w16, b16)
    v = out[:, :3364, :].reshape(n, 58, 58, 512)[:, :56, :56, :]
    v = v.reshape(n, 56, 56, 2, 2, 128).transpose(0, 1, 3, 2, 4, 5)
    return v.reshape(n, 112, 112, 128)


def _dt2(x4, w, b):
    n = x4.shape[0]
    xp = jnp.pad(x4, ((0, 0), (1, 1), (1, 1), (0, 0))).reshape(n, 12996, 128)
    xf = jnp.pad(xp, ((0, 0), (0, 4), (0, 0)))
    amap = {0: [(0, 3)], 1: [(0, 1), (1, 2)], 2: [(1, 0)]}
    w9 = jnp.zeros((9, 128, 12), F32)
    for jy in range(3):
        for jx in range(3):
            t = jy * 3 + jx
            for (a, kh) in amap[jy]:
                for (bb, kw) in amap[jx]:
                    c0 = (a * 2 + bb) * 3
                    w9 = w9.at[t, :, c0:c0 + 3].set(w[:, :, kh, kw])
    b12 = jnp.tile(b, 4).reshape(1, 12)
    offs = tuple(jy * 114 + jx for jy in range(3) for jx in range(3))
    out = pl.pallas_call(
        functools.partial(_shifted_mm_body, offs, 1824),
        grid=(n,),
        in_specs=[pl.BlockSpec((1, 13000, 128), lambda i: (i, 0, 0)),
                  _full((9, 128, 12)), _full((1, 12))],
        out_specs=pl.BlockSpec((1, 12768, 12), lambda i: (i, 0, 0)),
        out_shape=jax.ShapeDtypeStruct((n, 12768, 12), F32),
    )(xf, w9, b12)
    v = out.reshape(n, 112, 114, 12)[:, :, :112, :]
    v = v.reshape(n, 112, 112, 2, 2, 3).transpose(0, 1, 3, 2, 4, 5)
    return v.reshape(n, 224, 224, 3)


def kernel(imgs, ec1_w, ec1_b, ec2_w, ec2_b, erb1_w1, erb1_g, erb1_b, erb1_w2,
           erb2_w1, erb2_g, erb2_b, erb2_w2, codebook, drb1_w1, drb1_g,
           drb1_b, drb1_w2, drb2_w1, drb2_g, drb2_b, drb2_w2, dt1_w, dt1_b,
           dt2_w, dt2_b):
    x = jnp.transpose(imgs, (0, 2, 3, 1))
    x = _ec1(x, ec1_w, ec1_b)
    x = _ec2(x, ec2_w, ec2_b)
    x = _resblock(x, erb1_w1, erb1_g, erb1_b, erb1_w2)
    ze = _resblock(x, erb2_w1, erb2_g, erb2_b, erb2_w2)
    enc4, vq_loss = _vq(ze, codebook)
    y = _resblock(enc4, drb1_w1, drb1_g, drb1_b, drb1_w2)
    y = _resblock(y, drb2_w1, drb2_g, drb2_b, drb2_w2)
    y = _dt1(y, dt1_w, dt1_b)
    dec = _dt2(y, dt2_w, dt2_b)
    encoded = jnp.transpose(enc4, (0, 3, 1, 2))
    decoded = jnp.transpose(dec, (0, 3, 1, 2))
    return (encoded, decoded, vq_loss)
